# baseline (speedup 1.0000x reference)
import functools

import jax
import jax.numpy as jnp
from jax import lax
from jax.experimental import pallas as pl
from jax.experimental.pallas import tpu as pltpu
from jax.experimental.pallas import tpu_sc as plsc

f32 = jnp.float32
i32 = jnp.int32

NN = 10000
EE = 320000
HID = 256
NCLS = 40
NP = 10112
RPT = 632
EP = 327680
K = 128
K1 = 64
NC, NS, L = 2, 16, 16
ER = EP // K
ER1 = EP // K1
NCH01 = ER1 // NS
SUP01 = 16
NCH2 = ER // (NC * NS)
SUP2 = 16
BN = 1000
GRID = NN // BN

_HIGH = lax.Precision.HIGHEST


def _dot(a, b):
    return jnp.dot(a, b, preferred_element_type=f32, precision=_HIGH)


def _pre0_body(x_ref, we_ref, be_ref, w0_ref, as_ref, ad_ref,
               h0_ref, xls_ref, a_s_ref, a_d_ref, ws_ref):
    h0 = _dot(x_ref[...], we_ref[...]) + be_ref[...]
    h0_ref[...] = h0
    xl = _dot(h0, w0_ref[...])
    xls_ref[0] = xl[:, :128]
    xls_ref[1] = xl[:, 128:]
    asv = _dot(xl, as_ref[...])
    adv = _dot(xl, ad_ref[...])
    a_s_ref[...] = asv
    a_d_ref[...] = adv
    z = asv + adv
    ws_ref[...] = jnp.exp(jnp.maximum(z, 0.2 * z))


def _gat_post(acc_ref, den_ref, ws_ref, xlp_ref, b_ref):
    acc = jnp.concatenate([acc_ref[0], acc_ref[1]], axis=-1)
    den8 = jnp.concatenate([den_ref[0][:, :4], den_ref[1][:, :4]],
                           axis=-1)
    wself = ws_ref[...]
    xlp = jnp.concatenate([xlp_ref[0], xlp_ref[1]], axis=-1)
    wrep = jnp.broadcast_to(wself[:, :, None], (BN, 8, 32)).reshape(BN, 256)
    drep = jnp.broadcast_to((den8 + wself)[:, :, None],
                            (BN, 8, 32)).reshape(BN, 256)
    return jnp.maximum((acc + wrep * xlp) / drep + b_ref[...], 0.0)


def _mid_body(acc_ref, den_ref, ws_ref, xlp_ref, b_ref, w_ref, as_ref, ad_ref,
              h_ref, xls_ref, a_s_ref, a_d_ref, wsn_ref):
    h = _gat_post(acc_ref, den_ref, ws_ref, xlp_ref, b_ref)
    h_ref[...] = h
    xl = _dot(h, w_ref[...])
    xls_ref[0] = xl[:, :128]
    xls_ref[1] = xl[:, 128:]
    asv = _dot(xl, as_ref[...])
    adv = _dot(xl, ad_ref[...])
    a_s_ref[...] = asv
    a_d_ref[...] = adv
    z = asv + adv
    wsn_ref[...] = jnp.exp(jnp.maximum(z, 0.2 * z))


def _pre2_body(acc_ref, den_ref, ws_ref, xlp_ref, b_ref, w2_ref, as2_ref,
               ad2_ref, h_ref, xlt_ref, adt_ref, ws2_ref):
    h = _gat_post(acc_ref, den_ref, ws_ref, xlp_ref, b_ref)
    h_ref[...] = h
    xl2 = _dot(h, w2_ref[...])
    as2 = _dot(xl2, as2_ref[...])
    ad2 = _dot(xl2, ad2_ref[...])
    z = as2 + ad2
    ws2_ref[...] = jnp.exp(jnp.maximum(z, 0.2 * z))
    one = jnp.ones((BN, 1), f32)
    xlt_ref[...] = jnp.concatenate(
        [xl2, one, jnp.zeros((BN, 7), f32), as2, jnp.zeros((BN, 15), f32)],
        axis=-1)
    adt_ref[...] = jnp.concatenate(
        [ad2, jnp.zeros((BN, 15), f32)], axis=-1)


def _fin_body(acc2_ref, xlt_ref, ws2_ref, b2_ref, out_ref):
    acc = acc2_ref[0] + acc2_ref[1]
    w = ws2_ref[...]
    num = acc[:, :NCLS] + w * xlt_ref[...][:, :NCLS]
    den = acc[:, NCLS:NCLS + 1] + w
    out_ref[...] = num / den + b2_ref[...]


def _row_spec(shape):
    nd = len(shape)
    return pl.BlockSpec(shape, lambda i: (i,) + (0,) * (nd - 1))


def _full_spec(shape):
    nd = len(shape)
    return pl.BlockSpec(shape, lambda i: (0,) * nd)


def _split_spec(ch):
    return pl.BlockSpec((2, BN, ch), lambda i: (0, i, 0))


_pre0 = pl.pallas_call(
    _pre0_body,
    grid=(GRID,),
    in_specs=[_row_spec((BN, 128)), _full_spec((128, 256)), _full_spec((256,)),
              _full_spec((256, 256)), _full_spec((256, 8)), _full_spec((256, 8))],
    out_specs=[_row_spec((BN, 256)), _split_spec(128), _row_spec((BN, 8)),
               _row_spec((BN, 8)), _row_spec((BN, 8))],
    out_shape=[jax.ShapeDtypeStruct((NN, 256), f32),
               jax.ShapeDtypeStruct((2, NN, 128), f32),
               jax.ShapeDtypeStruct((NN, 8), f32),
               jax.ShapeDtypeStruct((NN, 8), f32),
               jax.ShapeDtypeStruct((NN, 8), f32)],
)

_mid = pl.pallas_call(
    _mid_body,
    grid=(GRID,),
    in_specs=[_split_spec(128), _split_spec(128), _row_spec((BN, 8)),
              _split_spec(128), _full_spec((256,)), _full_spec((256, 256)),
              _full_spec((256, 8)), _full_spec((256, 8))],
    out_specs=[_row_spec((BN, 256)), _split_spec(128), _row_spec((BN, 8)),
               _row_spec((BN, 8)), _row_spec((BN, 8))],
    out_shape=[jax.ShapeDtypeStruct((NN, 256), f32),
               jax.ShapeDtypeStruct((2, NN, 128), f32),
               jax.ShapeDtypeStruct((NN, 8), f32),
               jax.ShapeDtypeStruct((NN, 8), f32),
               jax.ShapeDtypeStruct((NN, 8), f32)],
)

_pre2 = pl.pallas_call(
    _pre2_body,
    grid=(GRID,),
    in_specs=[_split_spec(128), _split_spec(128), _row_spec((BN, 8)),
              _split_spec(128), _full_spec((256,)), _full_spec((256, NCLS)),
              _full_spec((NCLS, 1)), _full_spec((NCLS, 1))],
    out_specs=[_row_spec((BN, 256)), _row_spec((BN, 64)), _row_spec((BN, 16)),
               _row_spec((BN, 1))],
    out_shape=[jax.ShapeDtypeStruct((NN, 256), f32),
               jax.ShapeDtypeStruct((NN, 64), f32),
               jax.ShapeDtypeStruct((NN, 16), f32),
               jax.ShapeDtypeStruct((NN, 1), f32)],
)

_fin = pl.pallas_call(
    _fin_body,
    grid=(GRID,),
    in_specs=[_split_spec(128), _row_spec((BN, 64)), _row_spec((BN, 1)),
              _full_spec((NCLS,))],
    out_specs=_row_spec((BN, NCLS)),
    out_shape=jax.ShapeDtypeStruct((NN, NCLS), f32),
)


_MESH = plsc.VectorSubcoreMesh(core_axis_name="c", subcore_axis_name="s",
                               num_cores=NC, num_subcores=NS)
_SC_PARAMS = pltpu.CompilerParams(needs_layout_passes=False,
                                  use_tc_tiling_on_sc=False)


def _sc01_body(src_hbm, dst_hbm, xl_hbm, at_hbm, zacc_hbm, zden_hbm,
               acc_out, den_out,
               acc_sp, den_sp, sidxs, didxs,
               asb0, asb1, asb2, adb0, adb1, adb2,
               wb0, wb1, wb2, mg0, mg1, mg2,
               gs0, gs1, gs2, ss0, ss1, ss2, isem):
    asb = [asb0, asb1, asb2]
    adb = [adb0, adb1, adb2]
    wbuf = [wb0, wb1, wb2]
    msg = [mg0, mg1, mg2]
    gsem = [gs0, gs1, gs2]
    ssem = [ss0, ss1, ss2]
    c = lax.axis_index("c")
    s = lax.axis_index("s")
    r0 = s * RPT
    rowb = s * NCH01
    iot = lax.iota(i32, L)
    nsup = NCH01 // SUP01

    pltpu.sync_copy(zacc_hbm, acc_sp.at[pl.ds(r0, RPT)])
    pltpu.sync_copy(zden_hbm, den_sp.at[pl.ds(r0, RPT)])
    plsc.subcore_barrier()

    def idx_rows(ch):
        slot = (ch // SUP01) & 1
        j = ch % SUP01
        return sidxs.at[slot, j], didxs.at[slot, j]

    def g_descs(ch, b):
        si, di = idx_rows(ch)
        return (pltpu.make_async_copy(at_hbm.at[si], asb[b], gsem[b]),
                pltpu.make_async_copy(at_hbm.at[di], adb[b], gsem[b]),
                pltpu.make_async_copy(xl_hbm.at[c].at[si], msg[b], gsem[b]))

    def s_descs(ch, b):
        _, di = idx_rows(ch)
        return (pltpu.make_async_copy(wbuf[b], den_sp.at[di], ssem[b]),
                pltpu.make_async_copy(msg[b], acc_sp.at[di], ssem[b]))

    def pf_descs(sup):
        slot = sup & 1
        row = rowb + sup * SUP01
        return (pltpu.make_async_copy(src_hbm.at[pl.ds(row, SUP01)],
                                      sidxs.at[slot], isem),
                pltpu.make_async_copy(dst_hbm.at[pl.ds(row, SUP01)],
                                      didxs.at[slot], isem))

    colS = (iot & 3) + 4 * c
    colD = colS + 8

    def compute(b):
        def q8(q, _):
            for u in range(8):
                e = q * 8 + u
                erow = iot * 0 + e
                a = (plsc.load_gather(asb[b], [erow, colS])
                     + plsc.load_gather(adb[b], [erow, colD]))
                a = jnp.maximum(a, 0.2 * a)
                wbuf[b][e, :] = jnp.exp(a)
            for u in range(8):
                e = q * 8 + u
                erow = iot * 0 + e
                for h in range(4):
                    wv = plsc.load_gather(wbuf[b], [erow, iot * 0 + h])
                    for j in range(2):
                        sl = pl.ds(h * 32 + j * 16, L)
                        msg[b][e, sl] = msg[b][e, sl] * wv
            return 0

        lax.fori_loop(0, K1 // 8, q8, 0)

    pltpu.sync_copy(src_hbm.at[pl.ds(rowb, SUP01)], sidxs.at[0])
    pltpu.sync_copy(dst_hbm.at[pl.ds(rowb, SUP01)], didxs.at[0])
    for d in g_descs(0, 0):
        d.start()

    def phase(p, b, b1):
        @pl.when(jnp.logical_and((p & (SUP01 - 1)) == 0, p < NCH01))
        def _():
            sup = (p // SUP01) + 1
            @pl.when(sup < nsup)
            def _():
                for d in pf_descs(sup):
                    d.start()
        @pl.when(jnp.logical_and((p & (SUP01 - 1)) == 1, p < NCH01))
        def _():
            sup = (p // SUP01) + 1
            @pl.when(sup < nsup)
            def _():
                for d in pf_descs(sup):
                    d.wait()
        @pl.when(jnp.logical_and(p >= 2, p < NCH01 + 2))
        def _():
            for d in s_descs(p - 2, b1):
                d.wait()
        @pl.when(p + 1 < NCH01)
        def _():
            for d in g_descs(p + 1, b1):
                d.start()
        @pl.when(p < NCH01)
        def _():
            for d in g_descs(p, b):
                d.wait()
            compute(b)
            for d in s_descs(p, b):
                d.start(add=True)

    def outer(cc, _):
        for u in range(3):
            phase(cc * 3 + u, u, (u + 1) % 3)
        return 0

    lax.fori_loop(0, (NCH01 + 4) // 3, outer, 0)
    plsc.subcore_barrier()
    pltpu.sync_copy(acc_sp.at[pl.ds(r0, RPT)], acc_out.at[c].at[pl.ds(r0, RPT)])
    pltpu.sync_copy(den_sp.at[pl.ds(r0, RPT)],
                    den_out.at[c, pl.ds(r0, RPT), pl.ds(0, 16)])


_sc01 = functools.partial(
    pl.kernel,
    out_type=(jax.ShapeDtypeStruct((NC, NP, 128), f32),
              jax.ShapeDtypeStruct((NC, NP, 128), f32)),
    mesh=_MESH,
    compiler_params=_SC_PARAMS,
    scratch_types=[
        pltpu.VMEM_SHARED((NP, 128), f32),
        pltpu.VMEM_SHARED((NP, 16), f32),
        pltpu.VMEM((2, SUP01, K1), i32),
        pltpu.VMEM((2, SUP01, K1), i32),
    ] + [pltpu.VMEM((K1, 16), f32)] * 9
      + [pltpu.VMEM((K1, 128), f32)] * 3
      + [pltpu.SemaphoreType.DMA] * 7,
)(_sc01_body)


def _sc2_body(src_hbm, dst_hbm, xlt_hbm, adt_hbm, zacc_hbm, acc_out,
              acc_sp, sidxs, didxs,
              adb0, adb1, adb2, mg0, mg1, mg2, wbuf,
              gs0, gs1, gs2, ss0, ss1, ss2, isem):
    adb = [adb0, adb1, adb2]
    msg = [mg0, mg1, mg2]
    gsem = [gs0, gs1, gs2]
    ssem = [ss0, ss1, ss2]
    c = lax.axis_index("c")
    s = lax.axis_index("s")
    r0 = s * RPT
    rowb = c * (ER // 2) + s * NCH2
    iot = lax.iota(i32, L)
    c48 = iot * 0 + 48
    c0 = iot * 0
    nsup = NCH2 // SUP2

    pltpu.sync_copy(zacc_hbm, acc_sp.at[pl.ds(r0, RPT)])
    plsc.subcore_barrier()

    def idx_rows(ch):
        slot = (ch // SUP2) & 1
        j = ch % SUP2
        return sidxs.at[slot, j], didxs.at[slot, j]

    def g_descs(ch, b):
        si, di = idx_rows(ch)
        return (pltpu.make_async_copy(adt_hbm.at[di], adb[b], gsem[b]),
                pltpu.make_async_copy(xlt_hbm.at[si], msg[b], gsem[b]))

    def s_descs(ch, b):
        _, di = idx_rows(ch)
        return (pltpu.make_async_copy(msg[b], acc_sp.at[di], ssem[b]),)

    def pf_descs(sup):
        slot = sup & 1
        row = rowb + sup * SUP2
        return (pltpu.make_async_copy(src_hbm.at[pl.ds(row, SUP2)],
                                      sidxs.at[slot], isem),
                pltpu.make_async_copy(dst_hbm.at[pl.ds(row, SUP2)],
                                      didxs.at[slot], isem))

    def compute(b):
        def wgrp(g, _):
            rows = g * L + iot
            a = (plsc.load_gather(msg[b], [rows, c48])
                 + plsc.load_gather(adb[b], [rows, c0]))
            a = jnp.maximum(a, 0.2 * a)
            wbuf[pl.ds(g * L, L)] = jnp.exp(a)
            return 0

        lax.fori_loop(0, K // L, wgrp, 0)

        def q4(q, _):
            for u in range(4):
                e = q * 4 + u
                wv = plsc.load_gather(wbuf, [iot * 0 + e])
                for j in range(4):
                    sl = pl.ds(j * 16, L)
                    msg[b][e, sl] = msg[b][e, sl] * wv
            return 0

        lax.fori_loop(0, K // 4, q4, 0)

    pltpu.sync_copy(src_hbm.at[pl.ds(rowb, SUP2)], sidxs.at[0])
    pltpu.sync_copy(dst_hbm.at[pl.ds(rowb, SUP2)], didxs.at[0])
    for d in g_descs(0, 0):
        d.start()

    def phase(p, b, b1):
        @pl.when(jnp.logical_and((p & (SUP2 - 1)) == 0, p < NCH2))
        def _():
            sup = (p // SUP2) + 1
            @pl.when(sup < nsup)
            def _():
                for d in pf_descs(sup):
                    d.start()
        @pl.when(jnp.logical_and((p & (SUP2 - 1)) == 1, p < NCH2))
        def _():
            sup = (p // SUP2) + 1
            @pl.when(sup < nsup)
            def _():
                for d in pf_descs(sup):
                    d.wait()
        @pl.when(jnp.logical_and(p >= 2, p < NCH2 + 2))
        def _():
            for d in s_descs(p - 2, b1):
                d.wait()
        @pl.when(p + 1 < NCH2)
        def _():
            for d in g_descs(p + 1, b1):
                d.start()
        @pl.when(p < NCH2)
        def _():
            for d in g_descs(p, b):
                d.wait()
            compute(b)
            for d in s_descs(p, b):
                d.start(add=True)

    def outer(cc, _):
        for u in range(3):
            phase(cc * 3 + u, u, (u + 1) % 3)
        return 0

    lax.fori_loop(0, (NCH2 + 4) // 3, outer, 0)
    plsc.subcore_barrier()
    pltpu.sync_copy(acc_sp.at[pl.ds(r0, RPT)],
                    acc_out.at[c, pl.ds(r0, RPT), pl.ds(0, 64)])


_sc2 = functools.partial(
    pl.kernel,
    out_type=jax.ShapeDtypeStruct((NC, NP, 128), f32),
    mesh=_MESH,
    compiler_params=_SC_PARAMS,
    scratch_types=[
        pltpu.VMEM_SHARED((NP, 64), f32),
        pltpu.VMEM((2, SUP2, K), i32),
        pltpu.VMEM((2, SUP2, K), i32),
    ] + [pltpu.VMEM((K, 16), f32)] * 3
      + [pltpu.VMEM((K, 64), f32)] * 3
      + [pltpu.VMEM((K,), f32)]
      + [pltpu.SemaphoreType.DMA] * 7,
)(_sc2_body)


def _blockdiag(a):
    return (jnp.eye(8, dtype=f32)[:, None, :] * a[:, :, None]).reshape(256, 8)


def _alpha_tab(a_s, a_d):
    return jnp.pad(jnp.concatenate([a_s, a_d], axis=1),
                   ((0, NP - NN), (0, 0)))


def kernel(x, edge_index, W_embed, b_embed, W0, a_src0, a_dst0, b0,
           W1, a_src1, a_dst1, b1, W2, a_src2, a_dst2, b2):
    pad = EP - EE
    src = jnp.concatenate([edge_index[0], jnp.zeros((pad,), i32)])
    dst = jnp.concatenate([edge_index[1], jnp.full((pad,), NN, i32)])
    src64, dst64 = src.reshape(ER1, K1), dst.reshape(ER1, K1)
    src128, dst128 = src.reshape(ER, K), dst.reshape(ER, K)

    As0, Ad0 = _blockdiag(a_src0), _blockdiag(a_dst0)
    As1, Ad1 = _blockdiag(a_src1), _blockdiag(a_dst1)

    zacc = jnp.zeros((RPT, 128), f32)
    zden = jnp.zeros((RPT, 16), f32)
    zacc2 = jnp.zeros((RPT, 64), f32)

    h0, xls0, as0, ad0, ws0 = _pre0(x, W_embed, b_embed, W0, As0, Ad0)
    acc0, den0 = _sc01(src64, dst64, xls0, _alpha_tab(as0, ad0), zacc, zden)
    h1, xls1, as1, ad1, ws1 = _mid(acc0, den0, ws0, xls0, b0, W1, As1, Ad1)
    acc1, den1 = _sc01(src64, dst64, xls1, _alpha_tab(as1, ad1), zacc, zden)
    h2, xlt2, adt2, ws2 = _pre2(acc1, den1, ws1, xls1, b1, W2,
                                a_src2.T, a_dst2.T)
    adt2p = jnp.pad(adt2, ((0, NP - NN), (0, 0)))
    acc2 = _sc2(src128, dst128, xlt2, adt2p, zacc2)
    h3 = _fin(acc2, xlt2, ws2, b2)
    return jnp.concatenate([h0, h1, h2, h3], axis=-1)


# speedup vs baseline: 1.3739x; 1.3739x over previous
import functools

import jax
import jax.numpy as jnp
from jax import lax
from jax.experimental import pallas as pl
from jax.experimental.pallas import tpu as pltpu
from jax.experimental.pallas import tpu_sc as plsc

f32 = jnp.float32
i32 = jnp.int32

NN = 10000
EE = 320000
HID = 256
NCLS = 40
NP = 10112
RPT = 632
EP = 327680
K = 128
K1 = 64
NC, NS, L = 2, 16, 16
ER = EP // K
ER1 = EP // K1
NCH01 = ER1 // NS
SUP01 = 16
NCH2 = ER // (NC * NS)
SUP2 = 16
BN = 1000
GRID = NN // BN

_HIGH = lax.Precision.HIGHEST


def _dot(a, b):
    return jnp.dot(a, b, preferred_element_type=f32, precision=_HIGH)


def _pre0_body(x_ref, we_ref, be_ref, w0_ref, as_ref, ad_ref,
               h0_ref, xls_ref, a_s_ref, a_d_ref, ws_ref):
    h0 = _dot(x_ref[...], we_ref[...]) + be_ref[...]
    h0_ref[...] = h0
    xl = _dot(h0, w0_ref[...])
    xls_ref[0] = xl[:, :128]
    xls_ref[1] = xl[:, 128:]
    asv = _dot(xl, as_ref[...])
    adv = _dot(xl, ad_ref[...])
    a_s_ref[...] = asv
    a_d_ref[...] = adv
    z = asv + adv
    ws_ref[...] = jnp.exp(jnp.maximum(z, 0.2 * z))


def _gat_post(acc_ref, den_ref, ws_ref, xlp_ref, b_ref):
    acc = jnp.concatenate([acc_ref[0], acc_ref[1]], axis=-1)
    den8 = jnp.concatenate([den_ref[0][:, :4], den_ref[1][:, :4]],
                           axis=-1)
    wself = ws_ref[...]
    xlp = jnp.concatenate([xlp_ref[0], xlp_ref[1]], axis=-1)
    wrep = jnp.broadcast_to(wself[:, :, None], (BN, 8, 32)).reshape(BN, 256)
    drep = jnp.broadcast_to((den8 + wself)[:, :, None],
                            (BN, 8, 32)).reshape(BN, 256)
    return jnp.maximum((acc + wrep * xlp) / drep + b_ref[...], 0.0)


def _mid_body(acc_ref, den_ref, ws_ref, xlp_ref, b_ref, w_ref, as_ref, ad_ref,
              h_ref, xls_ref, a_s_ref, a_d_ref, wsn_ref):
    h = _gat_post(acc_ref, den_ref, ws_ref, xlp_ref, b_ref)
    h_ref[...] = h
    xl = _dot(h, w_ref[...])
    xls_ref[0] = xl[:, :128]
    xls_ref[1] = xl[:, 128:]
    asv = _dot(xl, as_ref[...])
    adv = _dot(xl, ad_ref[...])
    a_s_ref[...] = asv
    a_d_ref[...] = adv
    z = asv + adv
    wsn_ref[...] = jnp.exp(jnp.maximum(z, 0.2 * z))


def _pre2_body(acc_ref, den_ref, ws_ref, xlp_ref, b_ref, w2_ref, as2_ref,
               ad2_ref, h_ref, xlt_ref, adt_ref, ws2_ref):
    h = _gat_post(acc_ref, den_ref, ws_ref, xlp_ref, b_ref)
    h_ref[...] = h
    xl2 = _dot(h, w2_ref[...])
    as2 = _dot(xl2, as2_ref[...])
    ad2 = _dot(xl2, ad2_ref[...])
    z = as2 + ad2
    ws2_ref[...] = jnp.exp(jnp.maximum(z, 0.2 * z))
    one = jnp.ones((BN, 1), f32)
    xlt_ref[...] = jnp.concatenate(
        [xl2, one, jnp.zeros((BN, 7), f32), as2, jnp.zeros((BN, 15), f32)],
        axis=-1)
    adt_ref[...] = jnp.concatenate(
        [ad2, jnp.zeros((BN, 15), f32)], axis=-1)


def _fin_body(acc2_ref, xlt_ref, ws2_ref, b2_ref, out_ref):
    acc = acc2_ref[0] + acc2_ref[1]
    w = ws2_ref[...]
    num = acc[:, :NCLS] + w * xlt_ref[...][:, :NCLS]
    den = acc[:, NCLS:NCLS + 1] + w
    out_ref[...] = num / den + b2_ref[...]


def _row_spec(shape):
    nd = len(shape)
    return pl.BlockSpec(shape, lambda i: (i,) + (0,) * (nd - 1))


def _full_spec(shape):
    nd = len(shape)
    return pl.BlockSpec(shape, lambda i: (0,) * nd)


def _split_spec(ch):
    return pl.BlockSpec((2, BN, ch), lambda i: (0, i, 0))


_pre0 = pl.pallas_call(
    _pre0_body,
    grid=(GRID,),
    in_specs=[_row_spec((BN, 128)), _full_spec((128, 256)), _full_spec((256,)),
              _full_spec((256, 256)), _full_spec((256, 8)), _full_spec((256, 8))],
    out_specs=[_row_spec((BN, 256)), _split_spec(128), _row_spec((BN, 8)),
               _row_spec((BN, 8)), _row_spec((BN, 8))],
    out_shape=[jax.ShapeDtypeStruct((NN, 256), f32),
               jax.ShapeDtypeStruct((2, NN, 128), f32),
               jax.ShapeDtypeStruct((NN, 8), f32),
               jax.ShapeDtypeStruct((NN, 8), f32),
               jax.ShapeDtypeStruct((NN, 8), f32)],
)

_mid = pl.pallas_call(
    _mid_body,
    grid=(GRID,),
    in_specs=[_split_spec(128), _split_spec(16), _row_spec((BN, 8)),
              _split_spec(128), _full_spec((256,)), _full_spec((256, 256)),
              _full_spec((256, 8)), _full_spec((256, 8))],
    out_specs=[_row_spec((BN, 256)), _split_spec(128), _row_spec((BN, 8)),
               _row_spec((BN, 8)), _row_spec((BN, 8))],
    out_shape=[jax.ShapeDtypeStruct((NN, 256), f32),
               jax.ShapeDtypeStruct((2, NN, 128), f32),
               jax.ShapeDtypeStruct((NN, 8), f32),
               jax.ShapeDtypeStruct((NN, 8), f32),
               jax.ShapeDtypeStruct((NN, 8), f32)],
)

_pre2 = pl.pallas_call(
    _pre2_body,
    grid=(GRID,),
    in_specs=[_split_spec(128), _split_spec(16), _row_spec((BN, 8)),
              _split_spec(128), _full_spec((256,)), _full_spec((256, NCLS)),
              _full_spec((NCLS, 1)), _full_spec((NCLS, 1))],
    out_specs=[_row_spec((BN, 256)), _row_spec((BN, 64)), _row_spec((BN, 16)),
               _row_spec((BN, 1))],
    out_shape=[jax.ShapeDtypeStruct((NN, 256), f32),
               jax.ShapeDtypeStruct((NN, 64), f32),
               jax.ShapeDtypeStruct((NN, 16), f32),
               jax.ShapeDtypeStruct((NN, 1), f32)],
)

_fin = pl.pallas_call(
    _fin_body,
    grid=(GRID,),
    in_specs=[_split_spec(64), _row_spec((BN, 64)), _row_spec((BN, 1)),
              _full_spec((NCLS,))],
    out_specs=_row_spec((BN, NCLS)),
    out_shape=jax.ShapeDtypeStruct((NN, NCLS), f32),
)


_MESH = plsc.VectorSubcoreMesh(core_axis_name="c", subcore_axis_name="s",
                               num_cores=NC, num_subcores=NS)
_SC_PARAMS = pltpu.CompilerParams(needs_layout_passes=False,
                                  use_tc_tiling_on_sc=False)


def _sc01_body(src_hbm, dst_hbm, xl_hbm, as_hbm, ad_hbm, zacc_hbm, zden_hbm,
               acc_out, den_out,
               acc_sp, den_sp, sidxs, didxs,
               asb0, asb1, asb2, adb0, adb1, adb2,
               wb0, wb1, wb2, mg0, mg1, mg2,
               gs0, gs1, gs2, ss0, ss1, ss2, isem):
    asb = [asb0, asb1, asb2]
    adb = [adb0, adb1, adb2]
    wbuf = [wb0, wb1, wb2]
    msg = [mg0, mg1, mg2]
    gsem = [gs0, gs1, gs2]
    ssem = [ss0, ss1, ss2]
    c = lax.axis_index("c")
    s = lax.axis_index("s")
    r0 = s * RPT
    rowb = s * NCH01
    iot = lax.iota(i32, L)
    nsup = NCH01 // SUP01

    pltpu.sync_copy(zacc_hbm, acc_sp.at[pl.ds(r0, RPT)])
    pltpu.sync_copy(zden_hbm, den_sp.at[pl.ds(r0, RPT)])
    plsc.subcore_barrier()

    def idx_rows(ch):
        slot = (ch // SUP01) & 1
        j = ch % SUP01
        return sidxs.at[slot, j], didxs.at[slot, j]

    def g_descs(ch, b):
        si, di = idx_rows(ch)
        return (pltpu.make_async_copy(as_hbm.at[c].at[si], asb[b], gsem[b]),
                pltpu.make_async_copy(ad_hbm.at[c].at[di], adb[b], gsem[b]),
                pltpu.make_async_copy(xl_hbm.at[c].at[si], msg[b], gsem[b]))

    def s_descs(ch, b):
        _, di = idx_rows(ch)
        return (pltpu.make_async_copy(wbuf[b], den_sp.at[di], ssem[b]),
                pltpu.make_async_copy(msg[b], acc_sp.at[di], ssem[b]))

    def pf_descs(sup):
        slot = sup & 1
        row = rowb + sup * SUP01
        return (pltpu.make_async_copy(src_hbm.at[pl.ds(row, SUP01)],
                                      sidxs.at[slot], isem),
                pltpu.make_async_copy(dst_hbm.at[pl.ds(row, SUP01)],
                                      didxs.at[slot], isem))

    def compute(b):
        @functools.partial(plsc.parallel_loop, 0, K1 // 8)
        def q8(q):
            for u in range(8):
                e = q * 8 + u
                a = asb[b][e, :] + adb[b][e, :]
                a = jnp.maximum(a, 0.2 * a)
                wbuf[b][e, :] = jnp.exp(a)
            for u in range(8):
                e = q * 8 + u
                erow = iot * 0 + e
                for h in range(4):
                    wv = plsc.load_gather(wbuf[b], [erow, iot * 0 + h])
                    for j in range(2):
                        sl = pl.ds(h * 32 + j * 16, L)
                        msg[b][e, sl] = msg[b][e, sl] * wv

    pltpu.sync_copy(src_hbm.at[pl.ds(rowb, SUP01)], sidxs.at[0])
    pltpu.sync_copy(dst_hbm.at[pl.ds(rowb, SUP01)], didxs.at[0])
    for d in g_descs(0, 0):
        d.start()

    def phase(p, b, b1):
        @pl.when(jnp.logical_and((p & (SUP01 - 1)) == 0, p < NCH01))
        def _():
            sup = (p // SUP01) + 1
            @pl.when(sup < nsup)
            def _():
                for d in pf_descs(sup):
                    d.start()
        @pl.when(jnp.logical_and((p & (SUP01 - 1)) == 1, p < NCH01))
        def _():
            sup = (p // SUP01) + 1
            @pl.when(sup < nsup)
            def _():
                for d in pf_descs(sup):
                    d.wait()
        @pl.when(jnp.logical_and(p >= 2, p < NCH01 + 2))
        def _():
            for d in s_descs(p - 2, b1):
                d.wait()
        @pl.when(p + 1 < NCH01)
        def _():
            for d in g_descs(p + 1, b1):
                d.start()
        @pl.when(p < NCH01)
        def _():
            for d in g_descs(p, b):
                d.wait()
            compute(b)
            for d in s_descs(p, b):
                d.start(add=True)

    def outer(cc, _):
        for u in range(3):
            phase(cc * 3 + u, u, (u + 1) % 3)
        return 0

    lax.fori_loop(0, (NCH01 + 4) // 3, outer, 0)
    plsc.subcore_barrier()
    pltpu.sync_copy(acc_sp.at[pl.ds(r0, RPT)], acc_out.at[c].at[pl.ds(r0, RPT)])
    pltpu.sync_copy(den_sp.at[pl.ds(r0, RPT)], den_out.at[c].at[pl.ds(r0, RPT)])


_sc01 = functools.partial(
    pl.kernel,
    out_type=(jax.ShapeDtypeStruct((NC, NP, 128), f32),
              jax.ShapeDtypeStruct((NC, NP, 16), f32)),
    mesh=_MESH,
    compiler_params=_SC_PARAMS,
    scratch_types=[
        pltpu.VMEM_SHARED((NP, 128), f32),
        pltpu.VMEM_SHARED((NP, 16), f32),
        pltpu.VMEM((2, SUP01, K1), i32),
        pltpu.VMEM((2, SUP01, K1), i32),
    ] + [pltpu.VMEM((K1, 16), f32)] * 9
      + [pltpu.VMEM((K1, 128), f32)] * 3
      + [pltpu.SemaphoreType.DMA] * 7,
)(_sc01_body)


def _sc2_body(src_hbm, dst_hbm, xlt_hbm, adt_hbm, zacc_hbm, acc_out,
              acc_sp, sidxs, didxs,
              adb0, adb1, adb2, mg0, mg1, mg2, wbuf,
              gs0, gs1, gs2, ss0, ss1, ss2, isem):
    adb = [adb0, adb1, adb2]
    msg = [mg0, mg1, mg2]
    gsem = [gs0, gs1, gs2]
    ssem = [ss0, ss1, ss2]
    c = lax.axis_index("c")
    s = lax.axis_index("s")
    r0 = s * RPT
    rowb = c * (ER // 2) + s * NCH2
    iot = lax.iota(i32, L)
    c48 = iot * 0 + 48
    c0 = iot * 0
    nsup = NCH2 // SUP2

    pltpu.sync_copy(zacc_hbm, acc_sp.at[pl.ds(r0, RPT)])
    plsc.subcore_barrier()

    def idx_rows(ch):
        slot = (ch // SUP2) & 1
        j = ch % SUP2
        return sidxs.at[slot, j], didxs.at[slot, j]

    def g_descs(ch, b):
        si, di = idx_rows(ch)
        return (pltpu.make_async_copy(adt_hbm.at[di], adb[b], gsem[b]),
                pltpu.make_async_copy(xlt_hbm.at[si], msg[b], gsem[b]))

    def s_descs(ch, b):
        _, di = idx_rows(ch)
        return (pltpu.make_async_copy(msg[b], acc_sp.at[di], ssem[b]),)

    def pf_descs(sup):
        slot = sup & 1
        row = rowb + sup * SUP2
        return (pltpu.make_async_copy(src_hbm.at[pl.ds(row, SUP2)],
                                      sidxs.at[slot], isem),
                pltpu.make_async_copy(dst_hbm.at[pl.ds(row, SUP2)],
                                      didxs.at[slot], isem))

    def compute(b):
        def wgrp(g, _):
            rows = g * L + iot
            a = (plsc.load_gather(msg[b], [rows, c48])
                 + plsc.load_gather(adb[b], [rows, c0]))
            a = jnp.maximum(a, 0.2 * a)
            wbuf[pl.ds(g * L, L)] = jnp.exp(a)
            return 0

        lax.fori_loop(0, K // L, wgrp, 0)

        def q4(q, _):
            for u in range(4):
                e = q * 4 + u
                wv = plsc.load_gather(wbuf, [iot * 0 + e])
                for j in range(4):
                    sl = pl.ds(j * 16, L)
                    msg[b][e, sl] = msg[b][e, sl] * wv
            return 0

        lax.fori_loop(0, K // 4, q4, 0)

    pltpu.sync_copy(src_hbm.at[pl.ds(rowb, SUP2)], sidxs.at[0])
    pltpu.sync_copy(dst_hbm.at[pl.ds(rowb, SUP2)], didxs.at[0])
    for d in g_descs(0, 0):
        d.start()

    def phase(p, b, b1):
        @pl.when(jnp.logical_and((p & (SUP2 - 1)) == 0, p < NCH2))
        def _():
            sup = (p // SUP2) + 1
            @pl.when(sup < nsup)
            def _():
                for d in pf_descs(sup):
                    d.start()
        @pl.when(jnp.logical_and((p & (SUP2 - 1)) == 1, p < NCH2))
        def _():
            sup = (p // SUP2) + 1
            @pl.when(sup < nsup)
            def _():
                for d in pf_descs(sup):
                    d.wait()
        @pl.when(jnp.logical_and(p >= 2, p < NCH2 + 2))
        def _():
            for d in s_descs(p - 2, b1):
                d.wait()
        @pl.when(p + 1 < NCH2)
        def _():
            for d in g_descs(p + 1, b1):
                d.start()
        @pl.when(p < NCH2)
        def _():
            for d in g_descs(p, b):
                d.wait()
            compute(b)
            for d in s_descs(p, b):
                d.start(add=True)

    def outer(cc, _):
        for u in range(3):
            phase(cc * 3 + u, u, (u + 1) % 3)
        return 0

    lax.fori_loop(0, (NCH2 + 4) // 3, outer, 0)
    plsc.subcore_barrier()
    pltpu.sync_copy(acc_sp.at[pl.ds(r0, RPT)], acc_out.at[c].at[pl.ds(r0, RPT)])


_sc2 = functools.partial(
    pl.kernel,
    out_type=jax.ShapeDtypeStruct((NC, NP, 64), f32),
    mesh=_MESH,
    compiler_params=_SC_PARAMS,
    scratch_types=[
        pltpu.VMEM_SHARED((NP, 64), f32),
        pltpu.VMEM((2, SUP2, K), i32),
        pltpu.VMEM((2, SUP2, K), i32),
    ] + [pltpu.VMEM((K, 16), f32)] * 3
      + [pltpu.VMEM((K, 64), f32)] * 3
      + [pltpu.VMEM((K,), f32)]
      + [pltpu.SemaphoreType.DMA] * 7,
)(_sc2_body)


def _blockdiag(a):
    return (jnp.eye(8, dtype=f32)[:, None, :] * a[:, :, None]).reshape(256, 8)


def _split_pad(a):
    t = a.reshape(NN, 2, 4).transpose(1, 0, 2)
    return jnp.pad(t, ((0, 0), (0, NP - NN), (0, 12)))


def kernel(x, edge_index, W_embed, b_embed, W0, a_src0, a_dst0, b0,
           W1, a_src1, a_dst1, b1, W2, a_src2, a_dst2, b2):
    pad = EP - EE
    src = jnp.concatenate([edge_index[0], jnp.zeros((pad,), i32)])
    dst = jnp.concatenate([edge_index[1], jnp.full((pad,), NN, i32)])
    src64, dst64 = src.reshape(ER1, K1), dst.reshape(ER1, K1)
    src128, dst128 = src.reshape(ER, K), dst.reshape(ER, K)

    As0, Ad0 = _blockdiag(a_src0), _blockdiag(a_dst0)
    As1, Ad1 = _blockdiag(a_src1), _blockdiag(a_dst1)

    zacc = jnp.zeros((RPT, 128), f32)
    zden = jnp.zeros((RPT, 16), f32)
    zacc2 = jnp.zeros((RPT, 64), f32)

    h0, xls0, as0, ad0, ws0 = _pre0(x, W_embed, b_embed, W0, As0, Ad0)
    acc0, den0 = _sc01(src64, dst64, xls0, _split_pad(as0), _split_pad(ad0),
                       zacc, zden)
    h1, xls1, as1, ad1, ws1 = _mid(acc0, den0, ws0, xls0, b0, W1, As1, Ad1)
    acc1, den1 = _sc01(src64, dst64, xls1, _split_pad(as1), _split_pad(ad1),
                       zacc, zden)
    h2, xlt2, adt2, ws2 = _pre2(acc1, den1, ws1, xls1, b1, W2,
                                a_src2.T, a_dst2.T)
    adt2p = jnp.pad(adt2, ((0, NP - NN), (0, 0)))
    acc2 = _sc2(src128, dst128, xlt2, adt2p, zacc2)
    h3 = _fin(acc2, xlt2, ws2, b2)
    return jnp.concatenate([h0, h1, h2, h3], axis=-1)
